# Initial kernel scaffold; baseline (speedup 1.0000x reference)
import jax
import jax.numpy as jnp
from jax.experimental import pallas as pl


def kernel(model_input, table):
    raise NotImplementedError("write your pallas kernel here")


# speedup vs baseline: 1.1023x; 1.1023x over previous
import functools

import jax
import jax.numpy as jnp
from jax import lax
from jax.experimental import pallas as pl
from jax.experimental.pallas import tpu as pltpu
from jax.experimental.pallas import tpu_sc as plsc

NC = 2
NS = 16
NW = NC * NS


@functools.cache
def _make_gather(n, v, d, chunk):
    n_per_w = n // NW
    n_chunks = n_per_w // chunk
    mesh = plsc.VectorSubcoreMesh(core_axis_name="c", subcore_axis_name="s")

    @functools.partial(
        pl.kernel,
        out_type=jax.ShapeDtypeStruct((n, d), jnp.float32),
        mesh=mesh,
        scratch_types=[
            pltpu.VMEM((chunk,), jnp.int32),
            pltpu.VMEM((chunk, d), jnp.float32),
            pltpu.SemaphoreType.DMA,
        ],
        compiler_params=pltpu.CompilerParams(use_tc_tiling_on_sc=False),
    )
    def gather_kernel(table_hbm, idx_hbm, out_hbm, idx_v, rows_v, sem):
        wid = lax.axis_index("s") * NC + lax.axis_index("c")
        base = pl.multiple_of(wid * n_per_w, chunk)

        def body(i, carry):
            off = pl.multiple_of(base + i * chunk, chunk)
            pltpu.sync_copy(idx_hbm.at[pl.ds(off, chunk)], idx_v)
            pltpu.async_copy(table_hbm.at[idx_v], rows_v, sem).wait()
            pltpu.sync_copy(rows_v, out_hbm.at[pl.ds(off, chunk)])
            return carry

        lax.fori_loop(0, n_chunks, body, 0, unroll=False)

    return gather_kernel


def kernel(model_input, table):
    b, h = model_input.shape
    v, d = table.shape
    n = b * h
    idx = model_input.reshape(n).astype(jnp.int32)
    out = _make_gather(n, v, d, 1600)(table, idx)
    return out.reshape(b, h, d)


# speedup vs baseline: 1.1091x; 1.0062x over previous
import functools

import jax
import jax.numpy as jnp
from jax import lax
from jax.experimental import pallas as pl
from jax.experimental.pallas import tpu as pltpu
from jax.experimental.pallas import tpu_sc as plsc

NC = 2
NS = 16
NW = NC * NS


@functools.cache
def _make_gather(n, v, d, chunk, nbuf):
    n_per_w = n // NW
    n_chunks = n_per_w // chunk
    n_groups = n_chunks // nbuf
    assert n_per_w % chunk == 0 and n_chunks % nbuf == 0
    mesh = plsc.VectorSubcoreMesh(core_axis_name="c", subcore_axis_name="s")

    @functools.partial(
        pl.kernel,
        out_type=jax.ShapeDtypeStruct((n, d), jnp.float32),
        mesh=mesh,
        scratch_types=[
            pltpu.VMEM((n_per_w,), jnp.int32),
            pltpu.VMEM((nbuf, chunk, d), jnp.float32),
            pltpu.SemaphoreType.DMA((nbuf,)),
            pltpu.SemaphoreType.DMA((nbuf,)),
        ],
        compiler_params=pltpu.CompilerParams(use_tc_tiling_on_sc=False),
    )
    def gather_kernel(table_hbm, idx_hbm, out_hbm, idx_full, rows_v, gsem, osem):
        wid = lax.axis_index("s") * NC + lax.axis_index("c")
        base = pl.multiple_of(wid * n_per_w, chunk)
        pltpu.sync_copy(idx_hbm.at[pl.ds(base, n_per_w)], idx_full)

        def gather_desc(g, b):
            off = pl.multiple_of(g * chunk, chunk)
            return pltpu.make_async_copy(
                table_hbm.at[idx_full.at[pl.ds(off, chunk)]],
                rows_v.at[b],
                gsem.at[b],
            )

        def out_desc(g, b):
            off = pl.multiple_of(base + g * chunk, chunk)
            return pltpu.make_async_copy(
                rows_v.at[b],
                out_hbm.at[pl.ds(off, chunk)],
                osem.at[b],
            )

        for b in range(nbuf):
            gather_desc(jnp.int32(b), b).start()

        def group(go, carry):
            for b in range(nbuf):
                g = go * nbuf + b
                gather_desc(g, b).wait()
                out_desc(g, b).start()

            @pl.when(go < n_groups - 1)
            def _():
                for b in range(nbuf):
                    g_next = (go + 1) * nbuf + b
                    out_desc(go * nbuf + b, b).wait()
                    gather_desc(g_next, b).start()

            return carry

        lax.fori_loop(0, n_groups, group, 0, unroll=False)

        for b in range(nbuf):
            out_desc(jnp.int32((n_groups - 1) * nbuf + b), b).wait()

    return gather_kernel


def kernel(model_input, table):
    b, h = model_input.shape
    v, d = table.shape
    n = b * h
    idx = model_input.reshape(n).astype(jnp.int32)
    out = _make_gather(n, v, d, 800, 4)(table, idx)
    return out.reshape(b, h, d)


# speedup vs baseline: 1.7738x; 1.5994x over previous
import functools

import jax
import jax.numpy as jnp
from jax import lax
from jax.experimental import pallas as pl
from jax.experimental.pallas import tpu as pltpu
from jax.experimental.pallas import tpu_sc as plsc

NC = 2
NS = 16
NW = NC * NS


@functools.cache
def _make_gather(b, h, v, d, nbuf):
    b_per_w = b // NW
    n_groups = b_per_w // nbuf
    assert b % NW == 0 and b_per_w % nbuf == 0
    mesh = plsc.VectorSubcoreMesh(core_axis_name="c", subcore_axis_name="s")

    @functools.partial(
        pl.kernel,
        out_type=jax.ShapeDtypeStruct((b, h, d), jnp.float32),
        mesh=mesh,
        scratch_types=[
            pltpu.VMEM((b_per_w, h), jnp.int32),
            pltpu.VMEM((nbuf, h, d), jnp.float32),
            pltpu.SemaphoreType.DMA((nbuf,)),
            pltpu.SemaphoreType.DMA((nbuf,)),
        ],
        compiler_params=pltpu.CompilerParams(use_tc_tiling_on_sc=False),
    )
    def gather_kernel(table_hbm, inp_hbm, out_hbm, idx_v, rows_v, gsem, osem):
        wid = lax.axis_index("s") * NC + lax.axis_index("c")
        wb = pl.multiple_of(wid * b_per_w, b_per_w)
        pltpu.sync_copy(inp_hbm.at[pl.ds(wb, b_per_w)], idx_v)

        def gather_desc(bi, buf):
            return pltpu.make_async_copy(
                table_hbm.at[idx_v.at[bi]],
                rows_v.at[buf],
                gsem.at[buf],
            )

        def out_desc(bi, buf):
            return pltpu.make_async_copy(
                rows_v.at[buf],
                out_hbm.at[wb + bi],
                osem.at[buf],
            )

        for buf in range(nbuf):
            gather_desc(jnp.int32(buf), buf).start()

        def group(go, carry):
            for buf in range(nbuf):
                bi = go * nbuf + buf
                gather_desc(bi, buf).wait()
                out_desc(bi, buf).start()

            @pl.when(go < n_groups - 1)
            def _():
                for buf in range(nbuf):
                    out_desc(go * nbuf + buf, buf).wait()
                    gather_desc((go + 1) * nbuf + buf, buf).start()

            return carry

        lax.fori_loop(0, n_groups, group, 0, unroll=False)

        for buf in range(nbuf):
            out_desc(jnp.int32((n_groups - 1) * nbuf + buf), buf).wait()

    return gather_kernel


def kernel(model_input, table):
    b, h = model_input.shape
    v, d = table.shape
    return _make_gather(b, h, v, d, 8)(table, model_input)
